# baseline (speedup 1.0000x reference)
import functools

import jax
import jax.numpy as jnp
from jax import lax
from jax.experimental import pallas as pl
from jax.experimental.pallas import tpu as pltpu
from jax.experimental.pallas import tpu_sc as plsc

B, N, FEAT, PDIM, M, K, CIN, COUT = 8, 2048, 253, 3, 512, 16, 256, 256
BM = B * M
NC, NS = 2, 16
NW = NC * NS
QPW = BM // NW
QC = 4
QPB = M // NW
NCHB = QPB // QC

_SC_MESH = plsc.VectorSubcoreMesh(core_axis_name="c", subcore_axis_name="s")


def _sc_wid():
    return lax.axis_index("s") * NC + lax.axis_index("c")


@functools.partial(
    pl.kernel,
    mesh=_SC_MESH,
    out_type=jax.ShapeDtypeStruct((BM, CIN), jnp.float32),
    scratch_types=[
        pltpu.VMEM((QPW,), jnp.int32),
        pltpu.VMEM((QPW, CIN), jnp.float32),
        pltpu.SemaphoreType.DMA,
    ],
)
def _sc_qgather(feat_hbm, sidx_hbm, q_hbm, idx_v, rows_v, sem):
    base = _sc_wid() * QPW
    pltpu.sync_copy(sidx_hbm.at[pl.ds(base, QPW)], idx_v)
    pltpu.async_copy(feat_hbm.at[idx_v], rows_v, sem).wait()
    pltpu.sync_copy(rows_v, q_hbm.at[pl.ds(base, QPW)])


@functools.partial(
    pl.kernel,
    mesh=_SC_MESH,
    out_type=jax.ShapeDtypeStruct((M, CIN), jnp.float32),
    scratch_types=[
        pltpu.VMEM((NCHB, QC * K), jnp.int32),
        pltpu.VMEM((2, QC * K, CIN), jnp.float32),
        pltpu.VMEM((QPB, CIN), jnp.float32),
        pltpu.SemaphoreType.DMA,
        pltpu.SemaphoreType.DMA,
    ],
)
def _sc_gather_avg(feat_hbm, idx_hbm, avg_hbm, idx_v, rows_v, out_v, sem0,
                   sem1):
    wid = _sc_wid()
    base = wid * QPB
    pltpu.sync_copy(idx_hbm.at[wid], idx_v)
    sems = [sem0, sem1]

    def _fire(ch, b):
        pltpu.async_copy(feat_hbm.at[idx_v.at[ch]], rows_v.at[b], sems[b])

    def _accum(ch, b):
        pltpu.make_async_copy(feat_hbm.at[idx_v.at[ch]], rows_v.at[b],
                              sems[b]).wait()
        for qi in range(QC):
            for c in range(CIN // 16):
                sl = pl.ds(c * 16, 16)
                acc = rows_v[b, qi * K, sl]
                for k in range(1, K):
                    acc = acc + rows_v[b, qi * K + k, sl]
                out_v[ch * QC + qi, sl] = acc * (1.0 / K)

    _fire(0, 0)

    def pair(g, _):
        c0 = 2 * g
        _fire(c0 + 1, 1)
        _accum(c0, 0)

        @pl.when(c0 + 2 < NCHB)
        def _():
            _fire(c0 + 2, 0)

        _accum(c0 + 1, 1)
        return 0

    lax.fori_loop(0, NCHB // 2, pair, 0)
    pltpu.sync_copy(out_v, avg_hbm.at[pl.ds(base, QPB)])


def _knn_idx_body(q_ref, post_ref, idx_ref):
    q = q_ref[0]
    pt = post_ref[0]
    dx = q[:, FEAT:FEAT + 1] - pt[0:1, :]
    dy = q[:, FEAT + 1:FEAT + 2] - pt[1:2, :]
    dz = q[:, FEAT + 2:FEAT + 3] - pt[2:3, :]
    d = (dx * dx + dy * dy) + dz * dz
    iota = lax.broadcasted_iota(jnp.int32, (M, N), 1)
    for k in range(K):
        j = jnp.argmin(d, axis=1).astype(jnp.int32).reshape(M, 1)
        idx_ref[0, :, k:k + 1] = j
        if k < K - 1:
            d = jnp.where(iota == j, jnp.inf, d)


def _mlp_body(avg_ref, w1t_ref, b1_ref, g1_ref, be1_ref, w2t_ref, b2_ref,
              g2_ref, be2_ref, out_ref):
    h = lax.dot_general(
        avg_ref[...], w1t_ref[...], (((1,), (0,)), ((), ())),
        precision=lax.Precision.HIGHEST, preferred_element_type=jnp.float32,
    ) + b1_ref[...]
    mu = jnp.mean(h, axis=0, keepdims=True)
    var = jnp.mean((h - mu) ** 2, axis=0, keepdims=True)
    h = (h - mu) / jnp.sqrt(var + 1e-5) * g1_ref[...] + be1_ref[...]
    h = jnp.maximum(h, 0.0)
    h = lax.dot_general(
        h, w2t_ref[...], (((1,), (0,)), ((), ())),
        precision=lax.Precision.HIGHEST, preferred_element_type=jnp.float32,
    ) + b2_ref[...]
    mu = jnp.mean(h, axis=0, keepdims=True)
    var = jnp.mean((h - mu) ** 2, axis=0, keepdims=True)
    out_ref[...] = (h - mu) / jnp.sqrt(var + 1e-5) * g2_ref[...] + be2_ref[...]


@jax.jit
def kernel(x, pos, sampled_idx, W1, b1, gamma1, beta1, W2, b2, gamma2, beta2):
    pos_t = jnp.transpose(pos, (0, 2, 1))
    feat = jnp.concatenate([x, pos], axis=-1).reshape(B * N, CIN)

    q = _sc_qgather(feat, sampled_idx).reshape(B, M, CIN)

    knn_call = pl.pallas_call(
        _knn_idx_body,
        grid=(1,),
        in_specs=[
            pl.BlockSpec((1, M, CIN), lambda b: (b, 0, 0)),
            pl.BlockSpec((1, PDIM, N), lambda b: (b, 0, 0)),
        ],
        out_specs=pl.BlockSpec((1, M, K), lambda b: (b, 0, 0)),
        out_shape=jax.ShapeDtypeStruct((1, M, K), jnp.int32),
    )

    avgs = []
    for b in range(B):
        idx_b = knn_call(q[b:b + 1], pos_t[b:b + 1])
        idxg = (idx_b.reshape(NW, NCHB, QC * K) + b * N)
        avgs.append(_sc_gather_avg(feat, idxg))
    avg = jnp.concatenate(avgs, axis=0)

    out = pl.pallas_call(
        _mlp_body,
        out_shape=jax.ShapeDtypeStruct((BM, COUT), jnp.float32),
    )(avg, W1.T, b1.reshape(1, COUT),
      gamma1.reshape(1, COUT), beta1.reshape(1, COUT), W2.T,
      b2.reshape(1, COUT), gamma2.reshape(1, COUT), beta2.reshape(1, COUT))

    return out.reshape(B, M, COUT)


# speedup vs baseline: 1.1429x; 1.1429x over previous
import functools

import jax
import jax.numpy as jnp
from jax import lax
from jax.experimental import pallas as pl
from jax.experimental.pallas import tpu as pltpu
from jax.experimental.pallas import tpu_sc as plsc

B, N, FEAT, PDIM, M, K, CIN, COUT = 8, 2048, 253, 3, 512, 16, 256, 256
BM = B * M
NC, NS = 2, 16
NW = NC * NS
QPW = BM // NW
QC = 4
QPB = M // NW
NCHB = QPB // QC

_SC_MESH = plsc.VectorSubcoreMesh(core_axis_name="c", subcore_axis_name="s")


def _sc_wid():
    return lax.axis_index("s") * NC + lax.axis_index("c")


@functools.partial(
    pl.kernel,
    mesh=_SC_MESH,
    out_type=jax.ShapeDtypeStruct((BM, CIN), jnp.float32),
    scratch_types=[
        pltpu.VMEM((QPW,), jnp.int32),
        pltpu.VMEM((QPW, CIN), jnp.float32),
        pltpu.SemaphoreType.DMA,
    ],
)
def _sc_qgather(feat_hbm, sidx_hbm, q_hbm, idx_v, rows_v, sem):
    base = _sc_wid() * QPW
    pltpu.sync_copy(sidx_hbm.at[pl.ds(base, QPW)], idx_v)
    pltpu.async_copy(feat_hbm.at[idx_v], rows_v, sem).wait()
    pltpu.sync_copy(rows_v, q_hbm.at[pl.ds(base, QPW)])


def _make_gather_avg(nq):
    qpw = nq // NW
    nch = qpw // QC

    @functools.partial(
        pl.kernel,
        mesh=_SC_MESH,
        out_type=jax.ShapeDtypeStruct((nq, CIN), jnp.float32),
        scratch_types=[
            pltpu.VMEM((nch, QC * K), jnp.int32),
            pltpu.VMEM((2, QC * K, CIN), jnp.float32),
            pltpu.VMEM((qpw, CIN), jnp.float32),
            pltpu.SemaphoreType.DMA,
            pltpu.SemaphoreType.DMA,
        ],
    )
    def gather_avg(feat_hbm, idx_hbm, avg_hbm, idx_v, rows_v, out_v, sem0,
                   sem1):
        wid = _sc_wid()
        base = wid * qpw
        pltpu.sync_copy(idx_hbm.at[wid], idx_v)
        sems = [sem0, sem1]

        def _fire(ch, b):
            pltpu.async_copy(feat_hbm.at[idx_v.at[ch]], rows_v.at[b], sems[b])

        def _accum(ch, b):
            pltpu.make_async_copy(feat_hbm.at[idx_v.at[ch]], rows_v.at[b],
                                  sems[b]).wait()
            for qi in range(QC):
                for c in range(CIN // 16):
                    sl = pl.ds(c * 16, 16)
                    acc = rows_v[b, qi * K, sl]
                    for k in range(1, K):
                        acc = acc + rows_v[b, qi * K + k, sl]
                    out_v[ch * QC + qi, sl] = acc * (1.0 / K)

        _fire(0, 0)

        def pair(g, _):
            c0 = 2 * g
            _fire(c0 + 1, 1)
            _accum(c0, 0)

            @pl.when(c0 + 2 < nch)
            def _():
                _fire(c0 + 2, 0)

            _accum(c0 + 1, 1)
            return 0

        lax.fori_loop(0, nch // 2, pair, 0)
        pltpu.sync_copy(out_v, avg_hbm.at[pl.ds(base, qpw)])

    return gather_avg


GRP = 2
_sc_gather_avg = _make_gather_avg(GRP * M)


def _knn_idx_body(q_ref, post_ref, idx_ref):
    q = q_ref[0]
    pt = post_ref[0]
    dx = q[:, FEAT:FEAT + 1] - pt[0:1, :]
    dy = q[:, FEAT + 1:FEAT + 2] - pt[1:2, :]
    dz = q[:, FEAT + 2:FEAT + 3] - pt[2:3, :]
    d = (dx * dx + dy * dy) + dz * dz
    iota = lax.broadcasted_iota(jnp.int32, (M, N), 1)
    for k in range(K):
        j = jnp.argmin(d, axis=1).astype(jnp.int32).reshape(M, 1)
        idx_ref[0, :, k:k + 1] = j
        if k < K - 1:
            d = jnp.where(iota == j, jnp.inf, d)


def _mlp_body(avg_ref, w1t_ref, b1_ref, g1_ref, be1_ref, w2t_ref, b2_ref,
              g2_ref, be2_ref, out_ref):
    h = lax.dot_general(
        avg_ref[...], w1t_ref[...], (((1,), (0,)), ((), ())),
        precision=lax.Precision.HIGHEST, preferred_element_type=jnp.float32,
    ) + b1_ref[...]
    mu = jnp.mean(h, axis=0, keepdims=True)
    var = jnp.mean((h - mu) ** 2, axis=0, keepdims=True)
    h = (h - mu) / jnp.sqrt(var + 1e-5) * g1_ref[...] + be1_ref[...]
    h = jnp.maximum(h, 0.0)
    h = lax.dot_general(
        h, w2t_ref[...], (((1,), (0,)), ((), ())),
        precision=lax.Precision.HIGHEST, preferred_element_type=jnp.float32,
    ) + b2_ref[...]
    mu = jnp.mean(h, axis=0, keepdims=True)
    var = jnp.mean((h - mu) ** 2, axis=0, keepdims=True)
    out_ref[...] = (h - mu) / jnp.sqrt(var + 1e-5) * g2_ref[...] + be2_ref[...]


@jax.jit
def kernel(x, pos, sampled_idx, W1, b1, gamma1, beta1, W2, b2, gamma2, beta2):
    pos_t = jnp.transpose(pos, (0, 2, 1))
    feat = jnp.concatenate([x, pos], axis=-1).reshape(B * N, CIN)

    q = _sc_qgather(feat, sampled_idx).reshape(B, M, CIN)

    knn_call = pl.pallas_call(
        _knn_idx_body,
        grid=(1,),
        in_specs=[
            pl.BlockSpec((1, M, CIN), lambda b: (b, 0, 0)),
            pl.BlockSpec((1, PDIM, N), lambda b: (b, 0, 0)),
        ],
        out_specs=pl.BlockSpec((1, M, K), lambda b: (b, 0, 0)),
        out_shape=jax.ShapeDtypeStruct((1, M, K), jnp.int32),
    )

    nchg = (GRP * M) // NW // QC
    avgs = []
    group_idx = []
    for b in range(B):
        idx_b = knn_call(q[b:b + 1], pos_t[b:b + 1])
        group_idx.append(idx_b.reshape(M * K) + b * N)
        if len(group_idx) == GRP:
            idxg = jnp.concatenate(group_idx).reshape(NW, nchg, QC * K)
            avgs.append(_sc_gather_avg(feat, idxg))
            group_idx = []
    avg = jnp.concatenate(avgs, axis=0)

    out = pl.pallas_call(
        _mlp_body,
        out_shape=jax.ShapeDtypeStruct((BM, COUT), jnp.float32),
    )(avg, W1.T, b1.reshape(1, COUT),
      gamma1.reshape(1, COUT), beta1.reshape(1, COUT), W2.T,
      b2.reshape(1, COUT), gamma2.reshape(1, COUT), beta2.reshape(1, COUT))

    return out.reshape(B, M, COUT)
